# baseline (speedup 1.0000x reference)
import functools

import jax
import jax.numpy as jnp
from jax import lax
from jax.experimental import pallas as pl
from jax.experimental.pallas import tpu as pltpu
from jax.experimental.pallas import tpu_sc as plsc

EMBED_DIM = 64
PAD_DIM = 128
HIST_PAD = 56
NUM_CORES = 2
NUM_SUBCORES = 16
NUM_WORKERS = NUM_CORES * NUM_SUBCORES
CHUNK_B = 8
N_BUF = 2
VBLK = 128


@functools.partial(jax.jit, static_argnames=("vocab",))
def _to_rows(table_t, last_flat, *, vocab):
  n_blk = vocab // VBLK
  rem = vocab - n_blk * VBLK
  blk_per_w = n_blk // NUM_WORKERS
  n_extra = n_blk - blk_per_w * NUM_WORKERS
  n_iter = blk_per_w + (1 if n_extra else 0)
  n_iter += n_iter % N_BUF
  mesh = plsc.VectorSubcoreMesh(core_axis_name="c", subcore_axis_name="s")

  @functools.partial(
      pl.kernel,
      out_type=jax.ShapeDtypeStruct((vocab * EMBED_DIM,), jnp.float32),
      mesh=mesh,
      scratch_types=[
          pltpu.VMEM((EMBED_DIM, VBLK), jnp.float32),
          pltpu.VMEM((EMBED_DIM, VBLK), jnp.float32),
          pltpu.VMEM((VBLK * EMBED_DIM,), jnp.float32),
          pltpu.VMEM((VBLK * EMBED_DIM,), jnp.float32),
          pltpu.VMEM((max(rem, 1) * EMBED_DIM,), jnp.float32),
          pltpu.SemaphoreType.DMA,
          pltpu.SemaphoreType.DMA,
          pltpu.SemaphoreType.DMA,
          pltpu.SemaphoreType.DMA,
      ],
      compiler_params=pltpu.CompilerParams(
          use_tc_tiling_on_sc=True, needs_layout_passes=False),
  )
  def k(tab_hbm, last_hbm, out_hbm, in0, in1, st0, st1, edge_v,
        isem0, isem1, osem0, osem1):
    wid = lax.axis_index("s") * NUM_CORES + lax.axis_index("c")
    start = wid * blk_per_w + jnp.minimum(wid, n_extra)
    count = blk_per_w + jnp.where(wid < n_extra, 1, 0)
    ins = (in0, in1)
    sts = (st0, st1)
    isems = (isem0, isem1)
    osems = (osem0, osem1)
    ci = lax.iota(jnp.int32, 16)
    cis = tuple(ci + 16 * q for q in range(EMBED_DIM // 16))

    if rem:
      @pl.when(wid == NUM_WORKERS - 1)
      def _():
        pltpu.sync_copy(last_hbm, edge_v)
        pltpu.sync_copy(
            edge_v, out_hbm.at[pl.ds(n_blk * VBLK * EMBED_DIM, rem * EMBED_DIM)])

    def in_copy(blk, r):
      return pltpu.make_async_copy(
          tab_hbm.at[:, pl.ds(blk * VBLK, VBLK)], ins[r], isems[r])

    def out_copy(blk, r):
      return pltpu.make_async_copy(
          sts[r], out_hbm.at[pl.ds(blk * VBLK * EMBED_DIM, VBLK * EMBED_DIM)],
          osems[r])

    for r in range(N_BUF):
      @pl.when(r < count)
      def _():
        in_copy(start + r, r).start()

    @pl.loop(0, n_iter, step=N_BUF)
    def _(g):
      for r in range(N_BUF):
        i = g + r

        @pl.when(i < count)
        def _():
          blk = start + i
          in_copy(blk, r).wait()

          @pl.when(i >= N_BUF)
          def _():
            out_copy(start + i - N_BUF, r).wait()

          @pl.loop(0, VBLK, unroll=8)
          def _(v):
            vs = jnp.full((16,), v, jnp.int32)
            base = v * EMBED_DIM
            for q in range(EMBED_DIM // 16):
              val = plsc.load_gather(ins[r], [cis[q], vs])
              sts[r][pl.ds(base + 16 * q, 16)] = val

          out_copy(blk, r).start()

          @pl.when(i + N_BUF < count)
          def _():
            in_copy(blk + N_BUF, r).start()

    for r in range(N_BUF):
      @pl.when((count > r) & (count % N_BUF == (r + 1) % N_BUF))
      def _():
        out_copy(start + count - 1, r).wait()

      @pl.when((count > 1) & (count % N_BUF == r % N_BUF))
      def _():
        out_copy(start + count - 2, r).wait()

  return k(table_t, last_flat)


@functools.partial(jax.jit, static_argnames=("batch", "hist", "b_per_w"))
def _gather(idx, rows_tab, *, batch, hist, b_per_w):
  mesh = plsc.VectorSubcoreMesh(core_axis_name="c", subcore_axis_name="s")
  rows_per_chunk = CHUNK_B * hist
  n_chunks = b_per_w // CHUNK_B

  @functools.partial(
      pl.kernel,
      out_type=jax.ShapeDtypeStruct((batch, HIST_PAD, PAD_DIM), jnp.float32),
      mesh=mesh,
      scratch_types=[
          pltpu.VMEM((b_per_w * hist,), jnp.int32),
          pltpu.VMEM((rows_per_chunk, EMBED_DIM), jnp.float32),
          pltpu.VMEM((rows_per_chunk, EMBED_DIM), jnp.float32),
          pltpu.SemaphoreType.DMA,
          pltpu.SemaphoreType.DMA,
      ],
      compiler_params=pltpu.CompilerParams(use_tc_tiling_on_sc=False),
  )
  def k(idx_hbm, table_hbm, out_hbm, idx_v, rows0, rows1, sem0, sem1):
    wid = lax.axis_index("s") * NUM_CORES + lax.axis_index("c")
    base_b = wid * b_per_w
    rows = (rows0, rows1)
    sems = (sem0, sem1)

    pltpu.sync_copy(idx_hbm.at[pl.ds(base_b * hist, b_per_w * hist)], idx_v)

    def gather(c, b):
      return pltpu.make_async_copy(
          table_hbm.at[idx_v.at[pl.ds(c * rows_per_chunk, rows_per_chunk)]],
          rows[b], sems[b])

    for b in range(N_BUF):
      gather(b, b).start()

    @pl.loop(0, n_chunks, step=N_BUF)
    def _(g):
      for b in range(N_BUF):
        c = g + b
        gather(c, b).wait()
        for j in range(CHUNK_B):
          pltpu.sync_copy(
              rows[b].at[pl.ds(j * hist, hist), :],
              out_hbm.at[base_b + c * CHUNK_B + j, pl.ds(0, hist),
                         pl.ds(0, EMBED_DIM)])
        nxt = c + N_BUF

        @pl.when(nxt < n_chunks)
        def _():
          gather(nxt, b).start()

  return k(idx, rows_tab)


def kernel(x, table):
  batch, hist = x.shape
  vocab = table.shape[0]
  b_per_w = batch // NUM_WORKERS
  idx = x.reshape(batch * hist).astype(jnp.int32)
  n_full = (vocab // VBLK) * VBLK
  last_flat = table[n_full:, :].reshape((vocab - n_full) * EMBED_DIM)
  flat = _to_rows(table.T, last_flat, vocab=vocab)
  rows_tab = flat.reshape(vocab, EMBED_DIM)
  out = _gather(idx, rows_tab, batch=batch, hist=hist, b_per_w=b_per_w)
  return out[:, :hist, :EMBED_DIM]


# speedup vs baseline: 1.0004x; 1.0004x over previous
import functools

import jax
import jax.numpy as jnp
from jax import lax
from jax.experimental import pallas as pl
from jax.experimental.pallas import tpu as pltpu
from jax.experimental.pallas import tpu_sc as plsc

EMBED_DIM = 64
PAD_DIM = 128
HIST_PAD = 56
NUM_CORES = 2
NUM_SUBCORES = 16
NUM_WORKERS = NUM_CORES * NUM_SUBCORES
CHUNK_B = 8
N_BUF = 2
VBLK = 128


@functools.partial(jax.jit, static_argnames=("vocab",))
def _to_rows(table_t, last_flat, *, vocab):
  n_blk = vocab // VBLK
  rem = vocab - n_blk * VBLK
  blk_per_w = n_blk // NUM_WORKERS
  n_extra = n_blk - blk_per_w * NUM_WORKERS
  n_iter = blk_per_w + (1 if n_extra else 0)
  n_iter += n_iter % N_BUF
  mesh = plsc.VectorSubcoreMesh(core_axis_name="c", subcore_axis_name="s")

  @functools.partial(
      pl.kernel,
      out_type=jax.ShapeDtypeStruct((vocab * EMBED_DIM,), jnp.float32),
      mesh=mesh,
      scratch_types=[
          pltpu.VMEM((EMBED_DIM, VBLK), jnp.float32),
          pltpu.VMEM((EMBED_DIM, VBLK), jnp.float32),
          pltpu.VMEM((VBLK * EMBED_DIM,), jnp.float32),
          pltpu.VMEM((VBLK * EMBED_DIM,), jnp.float32),
          pltpu.VMEM((max(rem, 1) * EMBED_DIM,), jnp.float32),
          pltpu.SemaphoreType.DMA,
          pltpu.SemaphoreType.DMA,
          pltpu.SemaphoreType.DMA,
          pltpu.SemaphoreType.DMA,
      ],
      compiler_params=pltpu.CompilerParams(
          use_tc_tiling_on_sc=True, needs_layout_passes=False,
          disable_bounds_checks=True),
  )
  def k(tab_hbm, last_hbm, out_hbm, in0, in1, st0, st1, edge_v,
        isem0, isem1, osem0, osem1):
    wid = lax.axis_index("s") * NUM_CORES + lax.axis_index("c")
    start = wid * blk_per_w + jnp.minimum(wid, n_extra)
    count = blk_per_w + jnp.where(wid < n_extra, 1, 0)
    ins = (in0, in1)
    sts = (st0, st1)
    isems = (isem0, isem1)
    osems = (osem0, osem1)
    ci = lax.iota(jnp.int32, 16)
    cis = tuple(ci + 16 * q for q in range(EMBED_DIM // 16))

    if rem:
      @pl.when(wid == NUM_WORKERS - 1)
      def _():
        pltpu.sync_copy(last_hbm, edge_v)
        pltpu.sync_copy(
            edge_v, out_hbm.at[pl.ds(n_blk * VBLK * EMBED_DIM, rem * EMBED_DIM)])

    def in_copy(blk, r):
      return pltpu.make_async_copy(
          tab_hbm.at[:, pl.ds(blk * VBLK, VBLK)], ins[r], isems[r])

    def out_copy(blk, r):
      return pltpu.make_async_copy(
          sts[r], out_hbm.at[pl.ds(blk * VBLK * EMBED_DIM, VBLK * EMBED_DIM)],
          osems[r])

    for r in range(N_BUF):
      @pl.when(r < count)
      def _():
        in_copy(start + r, r).start()

    @pl.loop(0, n_iter, step=N_BUF)
    def _(g):
      for r in range(N_BUF):
        i = g + r

        @pl.when(i < count)
        def _():
          blk = start + i
          in_copy(blk, r).wait()

          @pl.when(i >= N_BUF)
          def _():
            out_copy(start + i - N_BUF, r).wait()

          @pl.loop(0, VBLK, unroll=8)
          def _(v):
            vs = jnp.full((16,), v, jnp.int32)
            base = v * EMBED_DIM
            for q in range(EMBED_DIM // 16):
              val = plsc.load_gather(ins[r], [cis[q], vs])
              sts[r][pl.ds(base + 16 * q, 16)] = val

          out_copy(blk, r).start()

          @pl.when(i + N_BUF < count)
          def _():
            in_copy(blk + N_BUF, r).start()

    for r in range(N_BUF):
      @pl.when((count > r) & (count % N_BUF == (r + 1) % N_BUF))
      def _():
        out_copy(start + count - 1, r).wait()

      @pl.when((count > 1) & (count % N_BUF == r % N_BUF))
      def _():
        out_copy(start + count - 2, r).wait()

  return k(table_t, last_flat)


@functools.partial(jax.jit, static_argnames=("batch", "hist", "b_per_w"))
def _gather(idx, rows_tab, *, batch, hist, b_per_w):
  mesh = plsc.VectorSubcoreMesh(core_axis_name="c", subcore_axis_name="s")
  rows_per_chunk = CHUNK_B * hist
  n_chunks = b_per_w // CHUNK_B

  @functools.partial(
      pl.kernel,
      out_type=jax.ShapeDtypeStruct((batch, HIST_PAD, PAD_DIM), jnp.float32),
      mesh=mesh,
      scratch_types=[
          pltpu.VMEM((b_per_w * hist,), jnp.int32),
          pltpu.VMEM((rows_per_chunk, EMBED_DIM), jnp.float32),
          pltpu.VMEM((rows_per_chunk, EMBED_DIM), jnp.float32),
          pltpu.SemaphoreType.DMA,
          pltpu.SemaphoreType.DMA,
      ],
      compiler_params=pltpu.CompilerParams(use_tc_tiling_on_sc=False),
  )
  def k(idx_hbm, table_hbm, out_hbm, idx_v, rows0, rows1, sem0, sem1):
    wid = lax.axis_index("s") * NUM_CORES + lax.axis_index("c")
    base_b = wid * b_per_w
    rows = (rows0, rows1)
    sems = (sem0, sem1)

    pltpu.sync_copy(idx_hbm.at[pl.ds(base_b * hist, b_per_w * hist)], idx_v)

    def gather(c, b):
      return pltpu.make_async_copy(
          table_hbm.at[idx_v.at[pl.ds(c * rows_per_chunk, rows_per_chunk)]],
          rows[b], sems[b])

    for b in range(N_BUF):
      gather(b, b).start()

    @pl.loop(0, n_chunks, step=N_BUF)
    def _(g):
      for b in range(N_BUF):
        c = g + b
        gather(c, b).wait()
        for j in range(CHUNK_B):
          pltpu.sync_copy(
              rows[b].at[pl.ds(j * hist, hist), :],
              out_hbm.at[base_b + c * CHUNK_B + j, pl.ds(0, hist),
                         pl.ds(0, EMBED_DIM)])
        nxt = c + N_BUF

        @pl.when(nxt < n_chunks)
        def _():
          gather(nxt, b).start()

  return k(idx, rows_tab)


def kernel(x, table):
  batch, hist = x.shape
  vocab = table.shape[0]
  b_per_w = batch // NUM_WORKERS
  idx = x.reshape(batch * hist).astype(jnp.int32)
  n_full = (vocab // VBLK) * VBLK
  last_flat = table[n_full:, :].reshape((vocab - n_full) * EMBED_DIM)
  flat = _to_rows(table.T, last_flat, vocab=vocab)
  rows_tab = flat.reshape(vocab, EMBED_DIM)
  out = _gather(idx, rows_tab, batch=batch, hist=hist, b_per_w=b_per_w)
  return out[:, :hist, :EMBED_DIM]


# speedup vs baseline: 1.9635x; 1.9628x over previous
import functools

import jax
import jax.numpy as jnp
from jax import lax
from jax.experimental import pallas as pl
from jax.experimental.pallas import tpu as pltpu
from jax.experimental.pallas import tpu_sc as plsc

EMBED_DIM = 64
PAD_DIM = 128
HIST_PAD = 56
NUM_CORES = 2
NUM_SUBCORES = 16
NUM_WORKERS = NUM_CORES * NUM_SUBCORES
CHUNK_B = 4
N_BUF = 4


@functools.partial(jax.jit, static_argnames=("batch", "hist", "b_per_w"))
def _gather(idx, table, *, batch, hist, b_per_w):
  mesh = plsc.VectorSubcoreMesh(core_axis_name="c", subcore_axis_name="s")
  rows_per_chunk = CHUNK_B * hist
  n_chunks = b_per_w // CHUNK_B

  @functools.partial(
      pl.kernel,
      out_type=jax.ShapeDtypeStruct((batch, HIST_PAD, PAD_DIM), jnp.float32),
      mesh=mesh,
      scratch_types=[
          pltpu.VMEM((b_per_w * hist,), jnp.int32),
          pltpu.VMEM((rows_per_chunk, EMBED_DIM), jnp.float32),
          pltpu.VMEM((rows_per_chunk, EMBED_DIM), jnp.float32),
          pltpu.VMEM((rows_per_chunk, EMBED_DIM), jnp.float32),
          pltpu.VMEM((rows_per_chunk, EMBED_DIM), jnp.float32),
          pltpu.SemaphoreType.DMA,
          pltpu.SemaphoreType.DMA,
          pltpu.SemaphoreType.DMA,
          pltpu.SemaphoreType.DMA,
      ],
      compiler_params=pltpu.CompilerParams(use_tc_tiling_on_sc=False),
  )
  def k(idx_hbm, table_hbm, out_hbm, idx_v, rows0, rows1, rows2, rows3,
        sem0, sem1, sem2, sem3):
    wid = lax.axis_index("s") * NUM_CORES + lax.axis_index("c")
    base_b = wid * b_per_w
    rows = (rows0, rows1, rows2, rows3)
    sems = (sem0, sem1, sem2, sem3)

    pltpu.sync_copy(idx_hbm.at[pl.ds(base_b * hist, b_per_w * hist)], idx_v)

    def gather(c, b):
      return pltpu.make_async_copy(
          table_hbm.at[idx_v.at[pl.ds(c * rows_per_chunk, rows_per_chunk)]],
          rows[b], sems[b])

    for b in range(N_BUF):
      gather(b, b).start()

    @pl.loop(0, n_chunks, step=N_BUF)
    def _(g):
      for b in range(N_BUF):
        c = g + b
        gather(c, b).wait()
        for j in range(CHUNK_B):
          pltpu.sync_copy(
              rows[b].at[pl.ds(j * hist, hist), :],
              out_hbm.at[base_b + c * CHUNK_B + j, pl.ds(0, hist),
                         pl.ds(0, EMBED_DIM)])
        nxt = c + N_BUF

        @pl.when(nxt < n_chunks)
        def _():
          gather(nxt, b).start()

  return k(idx, table)


def kernel(x, table):
  batch, hist = x.shape
  b_per_w = batch // NUM_WORKERS
  idx = x.reshape(batch * hist).astype(jnp.int32)
  out = _gather(idx, table, batch=batch, hist=hist, b_per_w=b_per_w)
  return out[:, :hist, :EMBED_DIM]
